# baseline (speedup 1.0000x reference)
import functools

import jax
import jax.numpy as jnp
from jax import lax
from jax.experimental import pallas as pl
from jax.experimental.pallas import tpu as pltpu
from jax.experimental.pallas import tpu_sc as plsc

VOCAB = 1000
BATCH = 4096
SEQ = 50

_info = plsc.get_sparse_core_info()
NC = _info.num_cores
NS = _info.num_subcores
NW = NC * NS
BG = BATCH // NW
W = 40
WP = 40
K = VOCAB // W
NVG = W // 8
N_CHUNKS = SEQ * K
NSEG = VOCAB * K
SROWS = 1568


def _emb_body(tab_hbm, idx_hbm, out_hbm,
              tab_sp, idx_v, si0, si1, segs0, segs1, xb0, xb1,
              g0, g1, w0, w1):
    c = lax.axis_index("c")
    s = lax.axis_index("s")
    wid = s * NC + c

    @pl.when(s < NS - 1)
    def _():
        pltpu.sync_copy(tab_hbm.at[pl.ds(s * SROWS, SROWS)],
                        tab_sp.at[pl.ds(s * SROWS, SROWS)])

    @pl.when(s == NS - 1)
    def _():
        pltpu.sync_copy(tab_hbm.at[pl.ds((NS - 1) * SROWS,
                                         NSEG - (NS - 1) * SROWS)],
                        tab_sp.at[pl.ds((NS - 1) * SROWS,
                                        NSEG - (NS - 1) * SROWS)])

    pltpu.sync_copy(idx_hbm.at[:, wid], idx_v)
    plsc.subcore_barrier()

    sis = (si0, si1)
    segss = (segs0, segs1)
    xbs = (xb0, xb1)
    gsems = (g0, g1)
    wsems = (w0, w1)

    lane = lax.iota(jnp.int32, 16)
    row_idx = tuple(lane + (g * 16) for g in range(8))

    def fill_seg_idx(i, b):
        ss = i // K
        kk = i % K
        for g in range(8):
            r = idx_v[ss, pl.ds(g * 16, 16)]
            sis[b][pl.ds(g * 16, 16)] = r * K + kk

    def gather_cp(b):
        return pltpu.make_async_copy(tab_sp.at[sis[b]], segss[b], gsems[b])

    def write_cp(i, b):
        ss = i // K
        kk = i % K
        return pltpu.make_async_copy(
            xbs[b], out_hbm.at[ss, pl.ds(kk * NVG, NVG), wid], wsems[b])

    def transpose(b):
        def vbody(vg, carry):
            vals = []
            for j in range(8):
                col = jnp.full((16,), vg * 8 + j, jnp.int32)
                for g in range(8):
                    vals.append(plsc.load_gather(segss[b],
                                                 [row_idx[g], col]))
            for j in range(8):
                for g in range(8):
                    xbs[b][vg, j, pl.ds(g * 16, 16)] = vals[j * 8 + g]
            return carry
        lax.fori_loop(0, NVG, vbody, 0)

    fill_seg_idx(0, 0)
    gather_cp(0).start()

    def step(jj, carry):
        for u in range(2):
            i = jj * 2 + u
            b = u
            other = 1 - u

            @pl.when(i + 1 < N_CHUNKS)
            def _():
                fill_seg_idx(i + 1, other)
                gather_cp(other).start()

            @pl.when(i >= 2)
            def _():
                write_cp(i - 2, b).wait()
            gather_cp(b).wait()
            transpose(b)
            write_cp(i, b).start()
        return carry

    lax.fori_loop(0, N_CHUNKS // 2, step, 0)
    write_cp(N_CHUNKS - 2, 0).wait()
    write_cp(N_CHUNKS - 1, 1).wait()


@jax.jit
def _embed(idx_t, tab_view):
    mesh = plsc.VectorSubcoreMesh(core_axis_name="c", subcore_axis_name="s")
    fn = pl.kernel(
        _emb_body,
        out_type=jax.ShapeDtypeStruct((SEQ, VOCAB // 8, NW, 8, BG),
                                      jnp.float32),
        mesh=mesh,
        scratch_types=[
            pltpu.VMEM_SHARED((NSEG, W), jnp.float32),
            pltpu.VMEM((SEQ, BG), jnp.int32),
            pltpu.VMEM((BG,), jnp.int32),
            pltpu.VMEM((BG,), jnp.int32),
            pltpu.VMEM((BG, WP), jnp.float32),
            pltpu.VMEM((BG, WP), jnp.float32),
            pltpu.VMEM((NVG, 8, BG), jnp.float32),
            pltpu.VMEM((NVG, 8, BG), jnp.float32),
            pltpu.SemaphoreType.DMA,
            pltpu.SemaphoreType.DMA,
            pltpu.SemaphoreType.DMA,
            pltpu.SemaphoreType.DMA,
        ],
        compiler_params=pltpu.CompilerParams(use_tc_tiling_on_sc=False,
                                             needs_layout_passes=False),
    )
    return fn(tab_view, idx_t)


def kernel(inputs, targets, table):
    idx_t = inputs.astype(jnp.int32).T.reshape(SEQ, NW, BG)
    tab_view = table.reshape(NSEG, W)
    x = _embed(idx_t, tab_view)
    return x.transpose(2, 4, 0, 1, 3).reshape(BATCH, SEQ, VOCAB)


# speedup vs baseline: 1.3573x; 1.3573x over previous
import functools

import jax
import jax.numpy as jnp
from jax import lax
from jax.experimental import pallas as pl
from jax.experimental.pallas import tpu as pltpu
from jax.experimental.pallas import tpu_sc as plsc

VOCAB = 1000
BATCH = 4096
SEQ = 50

_info = plsc.get_sparse_core_info()
NC = _info.num_cores
NS = _info.num_subcores
NW = NC * NS
BG = BATCH // NW
W = 200
WP = 200
K = VOCAB // W
NVG = W // 8
N_CHUNKS = SEQ * K
NSEG = VOCAB * K


def _emb_body(tab_hbm, idx_hbm, out_hbm,
              idx_v, si0, si1, segs0, segs1, xb0, xb1,
              g0, g1, w0, w1):
    c = lax.axis_index("c")
    s = lax.axis_index("s")
    wid = s * NC + c

    pltpu.sync_copy(idx_hbm.at[:, wid], idx_v)

    sis = (si0, si1)
    segss = (segs0, segs1)
    xbs = (xb0, xb1)
    gsems = (g0, g1)
    wsems = (w0, w1)

    lane = lax.iota(jnp.int32, 16)
    row_idx = tuple(lane + (g * 16) for g in range(8))

    def fill_seg_idx(i, b):
        ss = i // K
        kk = i % K
        for g in range(8):
            r = idx_v[ss, pl.ds(g * 16, 16)]
            sis[b][pl.ds(g * 16, 16)] = r * K + kk

    def gather_cp(b):
        return pltpu.make_async_copy(tab_hbm.at[sis[b]], segss[b], gsems[b])

    def write_cp(i, b):
        ss = i // K
        kk = i % K
        return pltpu.make_async_copy(
            xbs[b], out_hbm.at[ss, pl.ds(kk * NVG, NVG), wid], wsems[b])

    def transpose(b):
        def vbody(vg, carry):
            for j0 in range(0, 8, 4):
                vals = []
                for j in range(j0, j0 + 4):
                    col = jnp.full((16,), vg * 8 + j, jnp.int32)
                    for g in range(8):
                        vals.append(plsc.load_gather(segss[b],
                                                     [row_idx[g], col]))
                for j in range(j0, j0 + 4):
                    for g in range(8):
                        xbs[b][vg, j, pl.ds(g * 16, 16)] = \
                            vals[(j - j0) * 8 + g]
            return carry
        lax.fori_loop(0, NVG, vbody, 0)

    fill_seg_idx(0, 0)
    gather_cp(0).start()

    def step(jj, carry):
        for u in range(2):
            i = jj * 2 + u
            b = u
            other = 1 - u

            @pl.when(i + 1 < N_CHUNKS)
            def _():
                fill_seg_idx(i + 1, other)
                gather_cp(other).start()

            @pl.when(i >= 2)
            def _():
                write_cp(i - 2, b).wait()
            gather_cp(b).wait()
            transpose(b)
            write_cp(i, b).start()
        return carry

    lax.fori_loop(0, N_CHUNKS // 2, step, 0)
    write_cp(N_CHUNKS - 2, 0).wait()
    write_cp(N_CHUNKS - 1, 1).wait()


@jax.jit
def _embed(idx_t, tab_view):
    mesh = plsc.VectorSubcoreMesh(core_axis_name="c", subcore_axis_name="s")
    fn = pl.kernel(
        _emb_body,
        out_type=jax.ShapeDtypeStruct((SEQ, VOCAB // 8, NW, 8, BG),
                                      jnp.float32),
        mesh=mesh,
        scratch_types=[
            pltpu.VMEM((SEQ, BG), jnp.int32),
            pltpu.VMEM((BG,), jnp.int32),
            pltpu.VMEM((BG,), jnp.int32),
            pltpu.VMEM((BG, WP), jnp.float32),
            pltpu.VMEM((BG, WP), jnp.float32),
            pltpu.VMEM((NVG, 8, BG), jnp.float32),
            pltpu.VMEM((NVG, 8, BG), jnp.float32),
            pltpu.SemaphoreType.DMA,
            pltpu.SemaphoreType.DMA,
            pltpu.SemaphoreType.DMA,
            pltpu.SemaphoreType.DMA,
        ],
        compiler_params=pltpu.CompilerParams(use_tc_tiling_on_sc=False,
                                             needs_layout_passes=False),
    )
    return fn(tab_view, idx_t)


def kernel(inputs, targets, table):
    idx_t = inputs.astype(jnp.int32).T.reshape(SEQ, NW, BG)
    tab_view = table.reshape(NSEG, W)
    x = _embed(idx_t, tab_view)
    return x.transpose(2, 4, 0, 1, 3).reshape(BATCH, SEQ, VOCAB)


# speedup vs baseline: 1.3987x; 1.0305x over previous
import functools

import jax
import jax.numpy as jnp
from jax import lax
from jax.experimental import pallas as pl
from jax.experimental.pallas import tpu as pltpu
from jax.experimental.pallas import tpu_sc as plsc

VOCAB = 1000
BATCH = 4096
SEQ = 50

_info = plsc.get_sparse_core_info()
NC = _info.num_cores
NS = _info.num_subcores
NW = NC * NS
BG = BATCH // NW
W = 200
WP = 200
K = VOCAB // W
NVG = W // 8
N_CHUNKS = SEQ * K
NSEG = VOCAB * K


def _emb_body(tab_hbm, idx_hbm, out_hbm,
              idx_v, si0, si1, segs0, segs1, xb0, xb1,
              g0, g1, w0, w1):
    c = lax.axis_index("c")
    s = lax.axis_index("s")
    wid = s * NC + c

    pltpu.sync_copy(idx_hbm.at[:, wid], idx_v)

    sis = (si0, si1)
    segss = (segs0, segs1)
    xbs = (xb0, xb1)
    gsems = (g0, g1)
    wsems = (w0, w1)

    lane = lax.iota(jnp.int32, 16)
    row_idx = tuple(lane + (g * 16) for g in range(8))

    def fill_seg_idx(i, b):
        ss = i // K
        kk = i % K
        for g in range(8):
            r = idx_v[ss, pl.ds(g * 16, 16)]
            sis[b][pl.ds(g * 16, 16)] = r * K + kk

    def gather_cp(b):
        return pltpu.make_async_copy(tab_hbm.at[sis[b]], segss[b], gsems[b])

    def write_cp(i, b):
        ss = i // K
        kk = i % K
        return pltpu.make_async_copy(
            xbs[b], out_hbm.at[ss, pl.ds(kk * NVG, NVG), wid], wsems[b])

    def transpose(b):
        def vbody(vg, carry):
            for j0 in range(0, 8, 2):
                vals = []
                for j in range(j0, j0 + 2):
                    col = jnp.full((16,), vg * 8 + j, jnp.int32)
                    for g in range(8):
                        vals.append(plsc.load_gather(segss[b],
                                                     [row_idx[g], col]))
                for j in range(j0, j0 + 2):
                    for g in range(8):
                        xbs[b][vg, j, pl.ds(g * 16, 16)] = \
                            vals[(j - j0) * 8 + g]
            return carry
        lax.fori_loop(0, NVG, vbody, 0)

    fill_seg_idx(0, 0)
    gather_cp(0).start()

    def step(jj, carry):
        for u in range(2):
            i = jj * 2 + u
            b = u
            other = 1 - u

            @pl.when(i + 1 < N_CHUNKS)
            def _():
                fill_seg_idx(i + 1, other)
                gather_cp(other).start()

            @pl.when(i >= 2)
            def _():
                write_cp(i - 2, b).wait()
            gather_cp(b).wait()
            transpose(b)
            write_cp(i, b).start()
        return carry

    lax.fori_loop(0, N_CHUNKS // 2, step, 0)
    write_cp(N_CHUNKS - 2, 0).wait()
    write_cp(N_CHUNKS - 1, 1).wait()


@jax.jit
def _embed(idx_t, tab_view):
    mesh = plsc.VectorSubcoreMesh(core_axis_name="c", subcore_axis_name="s")
    fn = pl.kernel(
        _emb_body,
        out_type=jax.ShapeDtypeStruct((SEQ, VOCAB // 8, NW, 8, BG),
                                      jnp.float32),
        mesh=mesh,
        scratch_types=[
            pltpu.VMEM((SEQ, BG), jnp.int32),
            pltpu.VMEM((BG,), jnp.int32),
            pltpu.VMEM((BG,), jnp.int32),
            pltpu.VMEM((BG, WP), jnp.float32),
            pltpu.VMEM((BG, WP), jnp.float32),
            pltpu.VMEM((NVG, 8, BG), jnp.float32),
            pltpu.VMEM((NVG, 8, BG), jnp.float32),
            pltpu.SemaphoreType.DMA,
            pltpu.SemaphoreType.DMA,
            pltpu.SemaphoreType.DMA,
            pltpu.SemaphoreType.DMA,
        ],
        compiler_params=pltpu.CompilerParams(use_tc_tiling_on_sc=False,
                                             needs_layout_passes=False),
    )
    return fn(tab_view, idx_t)


def kernel(inputs, targets, table):
    idx_t = inputs.astype(jnp.int32).T.reshape(SEQ, NW, BG)
    tab_view = table.reshape(NSEG, W)
    x = _embed(idx_t, tab_view)
    return x.transpose(2, 4, 0, 1, 3).reshape(BATCH, SEQ, VOCAB)
